# Initial kernel scaffold; baseline (speedup 1.0000x reference)
import functools

import jax
import jax.numpy as jnp
from jax import lax
from jax.experimental import pallas as pl
from jax.experimental.pallas import tpu as pltpu
from jax.experimental.pallas import tpu_sc as plsc

N = 10000
D = 128
D_OUT = 40
NW = 32
NCORES = 2
NSUB = 16
CH = 128
NCHUNK = 80
EPAD = NW * NCHUNK * CH
NPAD = 10048
ROWS_PER_TILE = NPAD // NSUB
DEGW = 16

BN = 1000
GRID = N // BN

_mesh = plsc.VectorSubcoreMesh(core_axis_name="c", subcore_axis_name="s")


@functools.partial(
    pl.kernel,
    mesh=_mesh,
    out_type=jax.ShapeDtypeStruct((NCORES, NPAD, DEGW), jnp.float32),
    scratch_types=[
        pltpu.VMEM((NCHUNK, CH), jnp.int32),
        pltpu.VMEM((CH, DEGW), jnp.float32),
        pltpu.VMEM_SHARED((NPAD, DEGW), jnp.float32),
    ],
)
def _sc_degree(dst_hbm, ones_hbm, zeros_hbm, out_hbm, dst_v, ones_v, acc):
    cid = lax.axis_index("c")
    sid = lax.axis_index("s")
    wid = cid * NSUB + sid
    pltpu.sync_copy(dst_hbm.at[wid], dst_v)
    pltpu.sync_copy(ones_hbm, ones_v)
    r0 = sid * ROWS_PER_TILE
    pltpu.sync_copy(zeros_hbm.at[pl.ds(r0, ROWS_PER_TILE)],
                    acc.at[pl.ds(r0, ROWS_PER_TILE)])
    plsc.subcore_barrier()

    @pl.loop(0, NCHUNK)
    def _(c):
        pltpu.sync_copy(ones_v, acc.at[dst_v.at[c]], add=True)

    plsc.subcore_barrier()
    pltpu.sync_copy(acc.at[pl.ds(r0, ROWS_PER_TILE)],
                    out_hbm.at[cid, pl.ds(r0, ROWS_PER_TILE)])


@functools.partial(
    pl.kernel,
    mesh=_mesh,
    out_type=jax.ShapeDtypeStruct((NCORES, NPAD, D), jnp.float32),
    scratch_types=[
        pltpu.VMEM((NCHUNK, CH), jnp.int32),
        pltpu.VMEM((NCHUNK, CH), jnp.int32),
        pltpu.VMEM((CH, D), jnp.float32),
        pltpu.VMEM_SHARED((NPAD, D), jnp.float32),
    ],
)
def _sc_scatter(y_hbm, src_hbm, dst_hbm, zeros_hbm, out_hbm,
                src_v, dst_v, buf, acc):
    cid = lax.axis_index("c")
    sid = lax.axis_index("s")
    wid = cid * NSUB + sid
    pltpu.sync_copy(src_hbm.at[wid], src_v)
    pltpu.sync_copy(dst_hbm.at[wid], dst_v)
    r0 = sid * ROWS_PER_TILE
    pltpu.sync_copy(zeros_hbm.at[pl.ds(r0, ROWS_PER_TILE)],
                    acc.at[pl.ds(r0, ROWS_PER_TILE)])
    plsc.subcore_barrier()

    @pl.loop(0, NCHUNK)
    def _(c):
        pltpu.sync_copy(y_hbm.at[src_v.at[c]], buf)
        pltpu.sync_copy(buf, acc.at[dst_v.at[c]], add=True)

    plsc.subcore_barrier()
    pltpu.sync_copy(acc.at[pl.ds(r0, ROWS_PER_TILE)],
                    out_hbm.at[cid, pl.ds(r0, ROWS_PER_TILE)])


_PREC = lax.Precision.HIGHEST


def _tc_first_body(x_ref, w_ref, degp_ref, y_ref, dis_ref):
    deg = degp_ref[0, :, 0:1] + degp_ref[1, :, 0:1] + 1.0
    dis = lax.rsqrt(deg)
    xw = jnp.dot(x_ref[...], w_ref[...],
                 preferred_element_type=jnp.float32, precision=_PREC)
    y_ref[...] = xw * dis
    dis_ref[...] = dis


def _tc_first(x, w, degp):
    return pl.pallas_call(
        _tc_first_body,
        grid=(GRID,),
        in_specs=[
            pl.BlockSpec((BN, D), lambda i: (i, 0)),
            pl.BlockSpec((D, D), lambda i: (0, 0)),
            pl.BlockSpec((NCORES, BN, DEGW), lambda i: (0, i, 0)),
        ],
        out_specs=[
            pl.BlockSpec((BN, D), lambda i: (i, 0)),
            pl.BlockSpec((BN, 1), lambda i: (i, 0)),
        ],
        out_shape=[
            jax.ShapeDtypeStruct((N, D), jnp.float32),
            jax.ShapeDtypeStruct((N, 1), jnp.float32),
        ],
    )(x, w, degp)


def _tc_mid_body(p_ref, y_ref, dis_ref, b_ref, w_ref, o_ref):
    s = p_ref[0] + p_ref[1] + y_ref[...]
    dis = dis_ref[...]
    h = jnp.maximum(dis * s + b_ref[...], 0.0)
    o_ref[...] = jnp.dot(h, w_ref[...],
                         preferred_element_type=jnp.float32,
                         precision=_PREC) * dis


def _tc_mid(p, y, dis, b, w):
    return pl.pallas_call(
        _tc_mid_body,
        grid=(GRID,),
        in_specs=[
            pl.BlockSpec((NCORES, BN, D), lambda i: (0, i, 0)),
            pl.BlockSpec((BN, D), lambda i: (i, 0)),
            pl.BlockSpec((BN, 1), lambda i: (i, 0)),
            pl.BlockSpec((1, D), lambda i: (0, 0)),
            pl.BlockSpec((D, D), lambda i: (0, 0)),
        ],
        out_specs=pl.BlockSpec((BN, D), lambda i: (i, 0)),
        out_shape=jax.ShapeDtypeStruct((N, D), jnp.float32),
    )(p, y, dis, b, w)


def _tc_final_body(p_ref, y_ref, dis_ref, b_ref, wp1_ref, bp1_ref,
                   wp2_ref, bp2_ref, o_ref):
    s = p_ref[0] + p_ref[1] + y_ref[...]
    h = jnp.maximum(dis_ref[...] * s + b_ref[...], 0.0)
    t = jnp.dot(h, wp1_ref[...],
                preferred_element_type=jnp.float32, precision=_PREC)
    t = t + bp1_ref[...]
    o = jnp.dot(t, wp2_ref[...],
                preferred_element_type=jnp.float32, precision=_PREC)
    o = o + bp2_ref[...]
    m = jnp.max(o, axis=1, keepdims=True)
    lse = jnp.log(jnp.sum(jnp.exp(o - m), axis=1, keepdims=True)) + m
    o_ref[...] = o - lse


def _tc_final(p, y, dis, b, wp1, bp1, wp2, bp2):
    return pl.pallas_call(
        _tc_final_body,
        grid=(GRID,),
        in_specs=[
            pl.BlockSpec((NCORES, BN, D), lambda i: (0, i, 0)),
            pl.BlockSpec((BN, D), lambda i: (i, 0)),
            pl.BlockSpec((BN, 1), lambda i: (i, 0)),
            pl.BlockSpec((1, D), lambda i: (0, 0)),
            pl.BlockSpec((D, D), lambda i: (0, 0)),
            pl.BlockSpec((1, D), lambda i: (0, 0)),
            pl.BlockSpec((D, D_OUT), lambda i: (0, 0)),
            pl.BlockSpec((1, D_OUT), lambda i: (0, 0)),
        ],
        out_specs=pl.BlockSpec((BN, D_OUT), lambda i: (i, 0)),
        out_shape=jax.ShapeDtypeStruct((N, D_OUT), jnp.float32),
    )(p, y, dis, b, wp1, bp1, wp2, bp2)


def kernel(x, edge_index, W1, b1, W2, b2, W3, b3, Wp1, bp1, Wp2, bp2):
    src = edge_index[0].astype(jnp.int32)
    dst = edge_index[1].astype(jnp.int32)
    e = src.shape[0]
    pad = EPAD - e
    src_p = jnp.concatenate([src, jnp.zeros((pad,), jnp.int32)])
    dst_p = jnp.concatenate([dst, jnp.full((pad,), N, jnp.int32)])
    src_p = src_p.reshape(NW, NCHUNK, CH)
    dst_p = dst_p.reshape(NW, NCHUNK, CH)

    zeros_d = jnp.zeros((NPAD, D), jnp.float32)
    zeros_g = jnp.zeros((NPAD, DEGW), jnp.float32)
    ones_g = jnp.ones((CH, DEGW), jnp.float32)

    degp = _sc_degree(dst_p, ones_g, zeros_g)
    y1, dis = _tc_first(x, W1, degp)
    p1 = _sc_scatter(y1, src_p, dst_p, zeros_d)
    y2 = _tc_mid(p1, y1, dis, b1.reshape(1, D), W2)
    p2 = _sc_scatter(y2, src_p, dst_p, zeros_d)
    y3 = _tc_mid(p2, y2, dis, b2.reshape(1, D), W3)
    p3 = _sc_scatter(y3, src_p, dst_p, zeros_d)
    out = _tc_final(p3, y3, dis, b3.reshape(1, D), Wp1,
                    bp1.reshape(1, D), Wp2, bp2.reshape(1, D_OUT))
    return out


# speedup vs baseline: 7.8111x; 7.8111x over previous
import functools

import jax
import jax.numpy as jnp
from jax import lax
from jax.experimental import pallas as pl
from jax.experimental.pallas import tpu as pltpu
from jax.experimental.pallas import tpu_sc as plsc

N = 10000
D = 128
D_OUT = 40
NW = 32
NCORES = 2
NSUB = 16
CH = 128
NCHUNK = 80
EPAD = NW * NCHUNK * CH
NPAD = 10240
ROWS_PER_TILE = NPAD // NSUB
DEGW = 128

BN = 1000
GRID = N // BN

_mesh = plsc.VectorSubcoreMesh(core_axis_name="c", subcore_axis_name="s")


@functools.partial(
    pl.kernel,
    mesh=_mesh,
    out_type=jax.ShapeDtypeStruct((NCORES, NPAD, DEGW), jnp.float32),
    scratch_types=[
        pltpu.VMEM((NCHUNK, CH), jnp.int32),
        pltpu.VMEM((CH, DEGW), jnp.float32),
        pltpu.VMEM_SHARED((NPAD, DEGW), jnp.float32),
    ],
)
def _sc_degree(dst_hbm, ones_hbm, zeros_hbm, out_hbm, dst_v, ones_v, acc):
    cid = lax.axis_index("c")
    sid = lax.axis_index("s")
    wid = cid * NSUB + sid
    pltpu.sync_copy(dst_hbm.at[wid], dst_v)
    pltpu.sync_copy(ones_hbm, ones_v)
    r0 = sid * ROWS_PER_TILE
    pltpu.sync_copy(zeros_hbm.at[pl.ds(r0, ROWS_PER_TILE)],
                    acc.at[pl.ds(r0, ROWS_PER_TILE)])
    plsc.subcore_barrier()

    @pl.loop(0, NCHUNK)
    def _(c):
        pltpu.sync_copy(ones_v, acc.at[dst_v.at[c]], add=True)

    plsc.subcore_barrier()
    pltpu.sync_copy(acc.at[pl.ds(r0, ROWS_PER_TILE)],
                    out_hbm.at[cid, pl.ds(r0, ROWS_PER_TILE)])


@functools.partial(
    pl.kernel,
    mesh=_mesh,
    out_type=jax.ShapeDtypeStruct((NCORES, NPAD, D), jnp.float32),
    scratch_types=[
        pltpu.VMEM((NCHUNK, CH), jnp.int32),
        pltpu.VMEM((NCHUNK, CH), jnp.int32),
        pltpu.VMEM((CH, D), jnp.float32),
        pltpu.VMEM_SHARED((NPAD, D), jnp.float32),
    ],
)
def _sc_scatter(y_hbm, src_hbm, dst_hbm, zeros_hbm, out_hbm,
                src_v, dst_v, buf, acc):
    cid = lax.axis_index("c")
    sid = lax.axis_index("s")
    wid = cid * NSUB + sid
    pltpu.sync_copy(src_hbm.at[wid], src_v)
    pltpu.sync_copy(dst_hbm.at[wid], dst_v)
    r0 = sid * ROWS_PER_TILE
    pltpu.sync_copy(zeros_hbm.at[pl.ds(r0, ROWS_PER_TILE)],
                    acc.at[pl.ds(r0, ROWS_PER_TILE)])
    plsc.subcore_barrier()

    @pl.loop(0, NCHUNK)
    def _(c):
        pltpu.sync_copy(y_hbm.at[src_v.at[c]], buf)
        pltpu.sync_copy(buf, acc.at[dst_v.at[c]], add=True)

    plsc.subcore_barrier()
    pltpu.sync_copy(acc.at[pl.ds(r0, ROWS_PER_TILE)],
                    out_hbm.at[cid, pl.ds(r0, ROWS_PER_TILE)])


_PREC = lax.Precision.HIGHEST


def _tc_first_body(x_ref, w_ref, degp_ref, y_ref, dis_ref):
    deg = degp_ref[0, :, 0:1] + degp_ref[1, :, 0:1] + 1.0
    dis = lax.rsqrt(deg)
    xw = jnp.dot(x_ref[...], w_ref[...],
                 preferred_element_type=jnp.float32, precision=_PREC)
    y_ref[...] = xw * dis
    dis_ref[...] = dis


def _tc_first(x, w, degp):
    return pl.pallas_call(
        _tc_first_body,
        grid=(GRID,),
        in_specs=[
            pl.BlockSpec((BN, D), lambda i: (i, 0)),
            pl.BlockSpec((D, D), lambda i: (0, 0)),
            pl.BlockSpec((NCORES, BN, DEGW), lambda i: (0, i, 0)),
        ],
        out_specs=[
            pl.BlockSpec((BN, D), lambda i: (i, 0)),
            pl.BlockSpec((BN, 1), lambda i: (i, 0)),
        ],
        out_shape=[
            jax.ShapeDtypeStruct((N, D), jnp.float32),
            jax.ShapeDtypeStruct((N, 1), jnp.float32),
        ],
    )(x, w, degp)


def _tc_mid_body(p_ref, y_ref, dis_ref, b_ref, w_ref, o_ref):
    s = p_ref[0] + p_ref[1] + y_ref[...]
    dis = dis_ref[...]
    h = jnp.maximum(dis * s + b_ref[...], 0.0)
    o_ref[...] = jnp.dot(h, w_ref[...],
                         preferred_element_type=jnp.float32,
                         precision=_PREC) * dis


def _tc_mid(p, y, dis, b, w):
    return pl.pallas_call(
        _tc_mid_body,
        grid=(GRID,),
        in_specs=[
            pl.BlockSpec((NCORES, BN, D), lambda i: (0, i, 0)),
            pl.BlockSpec((BN, D), lambda i: (i, 0)),
            pl.BlockSpec((BN, 1), lambda i: (i, 0)),
            pl.BlockSpec((1, D), lambda i: (0, 0)),
            pl.BlockSpec((D, D), lambda i: (0, 0)),
        ],
        out_specs=pl.BlockSpec((BN, D), lambda i: (i, 0)),
        out_shape=jax.ShapeDtypeStruct((N, D), jnp.float32),
    )(p, y, dis, b, w)


def _tc_final_body(p_ref, y_ref, dis_ref, b_ref, wp1_ref, bp1_ref,
                   wp2_ref, bp2_ref, o_ref):
    s = p_ref[0] + p_ref[1] + y_ref[...]
    h = jnp.maximum(dis_ref[...] * s + b_ref[...], 0.0)
    t = jnp.dot(h, wp1_ref[...],
                preferred_element_type=jnp.float32, precision=_PREC)
    t = t + bp1_ref[...]
    o = jnp.dot(t, wp2_ref[...],
                preferred_element_type=jnp.float32, precision=_PREC)
    o = o + bp2_ref[...]
    m = jnp.max(o, axis=1, keepdims=True)
    lse = jnp.log(jnp.sum(jnp.exp(o - m), axis=1, keepdims=True)) + m
    o_ref[...] = o - lse


def _tc_final(p, y, dis, b, wp1, bp1, wp2, bp2):
    return pl.pallas_call(
        _tc_final_body,
        grid=(GRID,),
        in_specs=[
            pl.BlockSpec((NCORES, BN, D), lambda i: (0, i, 0)),
            pl.BlockSpec((BN, D), lambda i: (i, 0)),
            pl.BlockSpec((BN, 1), lambda i: (i, 0)),
            pl.BlockSpec((1, D), lambda i: (0, 0)),
            pl.BlockSpec((D, D), lambda i: (0, 0)),
            pl.BlockSpec((1, D), lambda i: (0, 0)),
            pl.BlockSpec((D, D_OUT), lambda i: (0, 0)),
            pl.BlockSpec((1, D_OUT), lambda i: (0, 0)),
        ],
        out_specs=pl.BlockSpec((BN, D_OUT), lambda i: (i, 0)),
        out_shape=jax.ShapeDtypeStruct((N, D_OUT), jnp.float32),
    )(p, y, dis, b, wp1, bp1, wp2, bp2)


def kernel(x, edge_index, W1, b1, W2, b2, W3, b3, Wp1, bp1, Wp2, bp2):
    src = edge_index[0].astype(jnp.int32)
    dst = edge_index[1].astype(jnp.int32)
    e = src.shape[0]
    pad = EPAD - e
    src_p = jnp.concatenate([src, jnp.zeros((pad,), jnp.int32)])
    dst_p = jnp.concatenate([dst, jnp.full((pad,), N, jnp.int32)])
    src_p = src_p.reshape(NW, NCHUNK, CH)
    dst_p = dst_p.reshape(NW, NCHUNK, CH)

    zeros_d = jnp.zeros((NPAD, D), jnp.float32)
    ones_g = jnp.ones((CH, DEGW), jnp.float32)

    degp = _sc_degree(dst_p, ones_g, zeros_d)
    y1, dis = _tc_first(x, W1, degp)
    p1 = _sc_scatter(y1, src_p, dst_p, zeros_d)
    y2 = _tc_mid(p1, y1, dis, b1.reshape(1, D), W2)
    p2 = _sc_scatter(y2, src_p, dst_p, zeros_d)
    y3 = _tc_mid(p2, y2, dis, b2.reshape(1, D), W3)
    p3 = _sc_scatter(y3, src_p, dst_p, zeros_d)
    out = _tc_final(p3, y3, dis, b3.reshape(1, D), Wp1,
                    bp1.reshape(1, D), Wp2, bp2.reshape(1, D_OUT))
    return out


# speedup vs baseline: 9.0116x; 1.1537x over previous
import functools

import jax
import jax.numpy as jnp
from jax import lax
from jax.experimental import pallas as pl
from jax.experimental.pallas import tpu as pltpu
from jax.experimental.pallas import tpu_sc as plsc

N = 10000
D = 128
D_OUT = 40
NW = 32
NCORES = 2
NSUB = 16
CH = 128
NCHUNK = 80
EPAD = NW * NCHUNK * CH
NPAD = 10240
ROWS_PER_TILE = NPAD // NSUB
DEGW = 128

BN = 1000
GRID = N // BN

_mesh = plsc.VectorSubcoreMesh(core_axis_name="c", subcore_axis_name="s")


@functools.partial(
    pl.kernel,
    mesh=_mesh,
    out_type=jax.ShapeDtypeStruct((NCORES, NPAD, DEGW), jnp.float32),
    scratch_types=[
        pltpu.VMEM((NCHUNK, CH), jnp.int32),
        pltpu.VMEM((CH, DEGW), jnp.float32),
        pltpu.VMEM_SHARED((NPAD, DEGW), jnp.float32),
    ],
)
def _sc_degree(dst_hbm, ones_hbm, zeros_hbm, out_hbm, dst_v, ones_v, acc):
    cid = lax.axis_index("c")
    sid = lax.axis_index("s")
    wid = cid * NSUB + sid
    pltpu.sync_copy(dst_hbm.at[wid], dst_v)
    pltpu.sync_copy(ones_hbm, ones_v)
    r0 = sid * ROWS_PER_TILE
    pltpu.sync_copy(zeros_hbm.at[pl.ds(r0, ROWS_PER_TILE)],
                    acc.at[pl.ds(r0, ROWS_PER_TILE)])
    plsc.subcore_barrier()

    @pl.loop(0, NCHUNK)
    def _(c):
        pltpu.sync_copy(ones_v, acc.at[dst_v.at[c]], add=True)

    plsc.subcore_barrier()
    pltpu.sync_copy(acc.at[pl.ds(r0, ROWS_PER_TILE)],
                    out_hbm.at[cid, pl.ds(r0, ROWS_PER_TILE)])


@functools.partial(
    pl.kernel,
    mesh=_mesh,
    out_type=jax.ShapeDtypeStruct((NCORES, NPAD, D), jnp.float32),
    scratch_types=[
        pltpu.VMEM((NCHUNK // 2, CH), jnp.int32),
        pltpu.VMEM((NCHUNK // 2, CH), jnp.int32),
        pltpu.VMEM((CH, D), jnp.float32),
        pltpu.VMEM((CH, D), jnp.float32),
        pltpu.VMEM_SHARED((NPAD, D), jnp.float32),
        pltpu.SemaphoreType.DMA,
        pltpu.SemaphoreType.DMA,
    ],
)
def _sc_scatter(y_hbm, src_hbm, dst_hbm, zeros_hbm, out_hbm,
                src_v, dst_v, bufa, bufb, acc, sema, semb):
    cid = lax.axis_index("c")
    sid = lax.axis_index("s")
    wid = cid * NSUB + sid
    half = NCHUNK // 2
    r0 = sid * ROWS_PER_TILE
    pltpu.sync_copy(zeros_hbm.at[pl.ds(r0, ROWS_PER_TILE)],
                    acc.at[pl.ds(r0, ROWS_PER_TILE)])
    plsc.subcore_barrier()

    for h in range(2):
        pltpu.sync_copy(src_hbm.at[wid, pl.ds(h * half, half)], src_v)
        pltpu.sync_copy(dst_hbm.at[wid, pl.ds(h * half, half)], dst_v)
        pltpu.async_copy(y_hbm.at[src_v.at[0]], bufa, sema)

        @pl.loop(0, half, step=2)
        def _(c):
            pltpu.async_copy(y_hbm.at[src_v.at[c + 1]], bufb, semb)
            pltpu.make_async_copy(y_hbm.at[src_v.at[c]], bufa, sema).wait()
            pltpu.sync_copy(bufa, acc.at[dst_v.at[c]], add=True)

            @pl.when(c + 2 < half)
            def _():
                pltpu.async_copy(y_hbm.at[src_v.at[c + 2]], bufa, sema)

            pltpu.make_async_copy(y_hbm.at[src_v.at[c + 1]], bufb, semb).wait()
            pltpu.sync_copy(bufb, acc.at[dst_v.at[c + 1]], add=True)

    plsc.subcore_barrier()
    pltpu.sync_copy(acc.at[pl.ds(r0, ROWS_PER_TILE)],
                    out_hbm.at[cid, pl.ds(r0, ROWS_PER_TILE)])


_PREC = lax.Precision.HIGHEST


def _tc_first_body(x_ref, w_ref, degp_ref, y_ref, dis_ref):
    deg = degp_ref[0, :, 0:1] + degp_ref[1, :, 0:1] + 1.0
    dis = lax.rsqrt(deg)
    xw = jnp.dot(x_ref[...], w_ref[...],
                 preferred_element_type=jnp.float32, precision=_PREC)
    y_ref[...] = xw * dis
    dis_ref[...] = dis


def _tc_first(x, w, degp):
    return pl.pallas_call(
        _tc_first_body,
        grid=(GRID,),
        in_specs=[
            pl.BlockSpec((BN, D), lambda i: (i, 0)),
            pl.BlockSpec((D, D), lambda i: (0, 0)),
            pl.BlockSpec((NCORES, BN, DEGW), lambda i: (0, i, 0)),
        ],
        out_specs=[
            pl.BlockSpec((BN, D), lambda i: (i, 0)),
            pl.BlockSpec((BN, 1), lambda i: (i, 0)),
        ],
        out_shape=[
            jax.ShapeDtypeStruct((N, D), jnp.float32),
            jax.ShapeDtypeStruct((N, 1), jnp.float32),
        ],
    )(x, w, degp)


def _tc_mid_body(p_ref, y_ref, dis_ref, b_ref, w_ref, o_ref):
    s = p_ref[0] + p_ref[1] + y_ref[...]
    dis = dis_ref[...]
    h = jnp.maximum(dis * s + b_ref[...], 0.0)
    o_ref[...] = jnp.dot(h, w_ref[...],
                         preferred_element_type=jnp.float32,
                         precision=_PREC) * dis


def _tc_mid(p, y, dis, b, w):
    return pl.pallas_call(
        _tc_mid_body,
        grid=(GRID,),
        in_specs=[
            pl.BlockSpec((NCORES, BN, D), lambda i: (0, i, 0)),
            pl.BlockSpec((BN, D), lambda i: (i, 0)),
            pl.BlockSpec((BN, 1), lambda i: (i, 0)),
            pl.BlockSpec((1, D), lambda i: (0, 0)),
            pl.BlockSpec((D, D), lambda i: (0, 0)),
        ],
        out_specs=pl.BlockSpec((BN, D), lambda i: (i, 0)),
        out_shape=jax.ShapeDtypeStruct((N, D), jnp.float32),
    )(p, y, dis, b, w)


def _tc_final_body(p_ref, y_ref, dis_ref, b_ref, wp1_ref, bp1_ref,
                   wp2_ref, bp2_ref, o_ref):
    s = p_ref[0] + p_ref[1] + y_ref[...]
    h = jnp.maximum(dis_ref[...] * s + b_ref[...], 0.0)
    t = jnp.dot(h, wp1_ref[...],
                preferred_element_type=jnp.float32, precision=_PREC)
    t = t + bp1_ref[...]
    o = jnp.dot(t, wp2_ref[...],
                preferred_element_type=jnp.float32, precision=_PREC)
    o = o + bp2_ref[...]
    m = jnp.max(o, axis=1, keepdims=True)
    lse = jnp.log(jnp.sum(jnp.exp(o - m), axis=1, keepdims=True)) + m
    o_ref[...] = o - lse


def _tc_final(p, y, dis, b, wp1, bp1, wp2, bp2):
    return pl.pallas_call(
        _tc_final_body,
        grid=(GRID,),
        in_specs=[
            pl.BlockSpec((NCORES, BN, D), lambda i: (0, i, 0)),
            pl.BlockSpec((BN, D), lambda i: (i, 0)),
            pl.BlockSpec((BN, 1), lambda i: (i, 0)),
            pl.BlockSpec((1, D), lambda i: (0, 0)),
            pl.BlockSpec((D, D), lambda i: (0, 0)),
            pl.BlockSpec((1, D), lambda i: (0, 0)),
            pl.BlockSpec((D, D_OUT), lambda i: (0, 0)),
            pl.BlockSpec((1, D_OUT), lambda i: (0, 0)),
        ],
        out_specs=pl.BlockSpec((BN, D_OUT), lambda i: (i, 0)),
        out_shape=jax.ShapeDtypeStruct((N, D_OUT), jnp.float32),
    )(p, y, dis, b, wp1, bp1, wp2, bp2)


def kernel(x, edge_index, W1, b1, W2, b2, W3, b3, Wp1, bp1, Wp2, bp2):
    src = edge_index[0].astype(jnp.int32)
    dst = edge_index[1].astype(jnp.int32)
    e = src.shape[0]
    pad = EPAD - e
    src_p = jnp.concatenate([src, jnp.zeros((pad,), jnp.int32)])
    dst_p = jnp.concatenate([dst, jnp.full((pad,), N, jnp.int32)])
    src_p = src_p.reshape(NW, NCHUNK, CH)
    dst_p = dst_p.reshape(NW, NCHUNK, CH)

    zeros_d = jnp.zeros((NPAD, D), jnp.float32)
    ones_g = jnp.ones((CH, DEGW), jnp.float32)

    degp = _sc_degree(dst_p, ones_g, zeros_d)
    y1, dis = _tc_first(x, W1, degp)
    p1 = _sc_scatter(y1, src_p, dst_p, zeros_d)
    y2 = _tc_mid(p1, y1, dis, b1.reshape(1, D), W2)
    p2 = _sc_scatter(y2, src_p, dst_p, zeros_d)
    y3 = _tc_mid(p2, y2, dis, b2.reshape(1, D), W3)
    p3 = _sc_scatter(y3, src_p, dst_p, zeros_d)
    out = _tc_final(p3, y3, dis, b3.reshape(1, D), Wp1,
                    bp1.reshape(1, D), Wp2, bp2.reshape(1, D_OUT))
    return out
